# Initial kernel scaffold; baseline (speedup 1.0000x reference)
import functools

import jax
import jax.numpy as jnp
from jax import lax
from jax.experimental import pallas as pl
from jax.experimental.pallas import tpu as pltpu
from jax.experimental.pallas import tpu_sc as plsc

N = 10000
E = 320000
D = 128
DEG_W = 16

NC = 2
NS = 16
NW = NC * NS
EPW = E // NW
C = 80
CHUNKS = EPW // C
RPT = N // NS


def _sc_segment_sum(src, dst, feat, zeros_agg, zeros_deg, ones_rows):
    mesh = plsc.VectorSubcoreMesh(core_axis_name="c", subcore_axis_name="s")

    @functools.partial(
        pl.kernel,
        out_type=(
            jax.ShapeDtypeStruct((NC, N, D), jnp.float32),
            jax.ShapeDtypeStruct((NC, N, DEG_W), jnp.float32),
        ),
        mesh=mesh,
        scratch_types=[
            pltpu.VMEM_SHARED((N, D), jnp.float32),
            pltpu.VMEM_SHARED((N, DEG_W), jnp.float32),
            pltpu.VMEM((C,), jnp.int32),
            pltpu.VMEM((C,), jnp.int32),
            pltpu.VMEM((C, D), jnp.float32),
            pltpu.VMEM((C, DEG_W), jnp.float32),
            pltpu.SemaphoreType.DMA,
        ],
    )
    def k(src_hbm, dst_hbm, feat_hbm, zagg_hbm, zdeg_hbm, ones_hbm,
          agg_out, deg_out, agg_sp, deg_sp, src_v, dst_v, rows_v, ones_v,
          sem):
        cid = lax.axis_index("c")
        sid = lax.axis_index("s")
        wid = sid * NC + cid

        pltpu.sync_copy(ones_hbm, ones_v)
        pltpu.sync_copy(zagg_hbm, agg_sp.at[pl.ds(sid * RPT, RPT)])
        pltpu.sync_copy(zdeg_hbm, deg_sp.at[pl.ds(sid * RPT, RPT)])
        plsc.subcore_barrier()

        def body(j, _):
            base = wid * EPW + j * C
            pltpu.sync_copy(src_hbm.at[pl.ds(base, C)], src_v)
            pltpu.sync_copy(dst_hbm.at[pl.ds(base, C)], dst_v)
            pltpu.async_copy(feat_hbm.at[src_v], rows_v, sem).wait()
            pltpu.sync_copy(rows_v, agg_sp.at[dst_v], add=True)
            pltpu.sync_copy(ones_v, deg_sp.at[dst_v], add=True)
            return 0

        lax.fori_loop(0, CHUNKS, body, 0)
        plsc.subcore_barrier()

        r0 = sid * RPT
        pltpu.sync_copy(agg_sp.at[pl.ds(r0, RPT)],
                        agg_out.at[cid, pl.ds(r0, RPT)])
        pltpu.sync_copy(deg_sp.at[pl.ds(r0, RPT)],
                        deg_out.at[cid, pl.ds(r0, RPT)])

    return k(src, dst, feat, zeros_agg, zeros_deg, ones_rows)


def _tc_finish_body(agg_ref, deg_ref, he_ref, w_ref, out_ref):
    x = agg_ref[0] + agg_ref[1]
    x = x * he_ref[...]
    y = jnp.dot(x, w_ref[...], preferred_element_type=jnp.float32)
    deg = deg_ref[0, :, 0:1] + deg_ref[1, :, 0:1]
    out_ref[...] = y * (1.0 / jnp.maximum(deg, 1.0))


def _tc_finish(agg, deg, h_e, W):
    bn = 2000
    grid = (N // bn,)
    return pl.pallas_call(
        _tc_finish_body,
        grid=grid,
        in_specs=[
            pl.BlockSpec((NC, bn, D), lambda i: (0, i, 0)),
            pl.BlockSpec((NC, bn, DEG_W), lambda i: (0, i, 0)),
            pl.BlockSpec((1, D), lambda i: (0, 0)),
            pl.BlockSpec((D, D), lambda i: (0, 0)),
        ],
        out_specs=pl.BlockSpec((bn, D), lambda i: (i, 0)),
        out_shape=jax.ShapeDtypeStruct((N, D), jnp.float32),
    )(agg, deg, h_e, W)


def kernel(feat, edge_index, h_e, W):
    src = edge_index[0]
    dst = edge_index[1]
    zeros_agg = jnp.zeros((RPT, D), jnp.float32)
    zeros_deg = jnp.zeros((RPT, DEG_W), jnp.float32)
    ones_rows = jnp.zeros((C, DEG_W), jnp.float32).at[:, 0].set(1.0)
    agg, deg = _sc_segment_sum(src, dst, feat, zeros_agg, zeros_deg,
                               ones_rows)
    return _tc_finish(agg, deg, h_e, W)


# speedup vs baseline: 5.0072x; 5.0072x over previous
import functools

import jax
import jax.numpy as jnp
from jax import lax
from jax.experimental import pallas as pl
from jax.experimental.pallas import tpu as pltpu
from jax.experimental.pallas import tpu_sc as plsc

N = 10000
NP = 10112
E = 320000
D = 128
DEG_W = 16

NC = 2
NS = 16
NW = NC * NS
EPW = E // NW
C = 80
CHUNKS = EPW // C
RPT = NP // NS
ZR = 79
ZCH = RPT // ZR


def _sc_segment_sum(src, dst, feat, zeros_agg, zeros_deg, ones_rows):
    mesh = plsc.VectorSubcoreMesh(core_axis_name="c", subcore_axis_name="s")

    @functools.partial(
        pl.kernel,
        out_type=(
            jax.ShapeDtypeStruct((NC, NP, D), jnp.float32),
            jax.ShapeDtypeStruct((NC, NP, DEG_W), jnp.float32),
        ),
        mesh=mesh,
        compiler_params=pltpu.CompilerParams(use_tc_tiling_on_sc=False),
        scratch_types=[
            pltpu.VMEM_SHARED((NP, D), jnp.float32),
            pltpu.VMEM_SHARED((NP, DEG_W), jnp.float32),
            pltpu.VMEM((C,), jnp.int32),
            pltpu.VMEM((C,), jnp.int32),
            pltpu.VMEM((C, D), jnp.float32),
            pltpu.VMEM((C, DEG_W), jnp.float32),
            pltpu.SemaphoreType.DMA,
        ],
    )
    def k(src_hbm, dst_hbm, feat_hbm, zagg_hbm, zdeg_hbm, ones_hbm,
          agg_out, deg_out, agg_sp, deg_sp, src_v, dst_v, rows_v, ones_v,
          sem):
        cid = lax.axis_index("c")
        sid = lax.axis_index("s")
        wid = sid * NC + cid
        r0 = sid * RPT

        pltpu.sync_copy(zagg_hbm, rows_v.at[pl.ds(0, ZR)])
        pltpu.sync_copy(zdeg_hbm, ones_v.at[pl.ds(0, ZR)])
        for m in range(ZCH):
            pltpu.sync_copy(rows_v.at[pl.ds(0, ZR)],
                            agg_sp.at[pl.ds(r0 + m * ZR, ZR)])
            pltpu.sync_copy(ones_v.at[pl.ds(0, ZR)],
                            deg_sp.at[pl.ds(r0 + m * ZR, ZR)])
        pltpu.sync_copy(ones_hbm, ones_v)
        plsc.subcore_barrier()

        def body(j, _):
            base = wid * EPW + j * C
            pltpu.sync_copy(src_hbm.at[pl.ds(base, C)], src_v)
            pltpu.sync_copy(dst_hbm.at[pl.ds(base, C)], dst_v)
            pltpu.async_copy(feat_hbm.at[src_v], rows_v, sem).wait()
            pltpu.sync_copy(rows_v, agg_sp.at[dst_v], add=True)
            pltpu.sync_copy(ones_v, deg_sp.at[dst_v], add=True)
            return 0

        lax.fori_loop(0, CHUNKS, body, 0)
        plsc.subcore_barrier()

        for m in range(ZCH):
            rr = r0 + m * ZR
            pltpu.sync_copy(agg_sp.at[pl.ds(rr, ZR)],
                            rows_v.at[pl.ds(0, ZR)])
            pltpu.sync_copy(rows_v.at[pl.ds(0, ZR)],
                            agg_out.at[cid, pl.ds(rr, ZR)])
            pltpu.sync_copy(deg_sp.at[pl.ds(rr, ZR)],
                            ones_v.at[pl.ds(0, ZR)])
            pltpu.sync_copy(ones_v.at[pl.ds(0, ZR)],
                            deg_out.at[cid, pl.ds(rr, ZR)])

    return k(src, dst, feat, zeros_agg, zeros_deg, ones_rows)


def _tc_finish_body(agg_ref, deg_ref, he_ref, w_ref, out_ref):
    x = agg_ref[0] + agg_ref[1]
    x = x * he_ref[...]
    y = jnp.dot(x, w_ref[...], preferred_element_type=jnp.float32)
    deg = deg_ref[0, :, 0:1] + deg_ref[1, :, 0:1]
    out_ref[...] = y * (1.0 / jnp.maximum(deg, 1.0))


def _tc_finish(agg, deg, h_e, W):
    bn = 2000
    grid = (N // bn,)
    return pl.pallas_call(
        _tc_finish_body,
        grid=grid,
        in_specs=[
            pl.BlockSpec((NC, bn, D), lambda i: (0, i, 0)),
            pl.BlockSpec((NC, bn, DEG_W), lambda i: (0, i, 0)),
            pl.BlockSpec((1, D), lambda i: (0, 0)),
            pl.BlockSpec((D, D), lambda i: (0, 0)),
        ],
        out_specs=pl.BlockSpec((bn, D), lambda i: (i, 0)),
        out_shape=jax.ShapeDtypeStruct((N, D), jnp.float32),
    )(agg, deg, h_e, W)


def kernel(feat, edge_index, h_e, W):
    src = edge_index[0]
    dst = edge_index[1]
    zeros_agg = jnp.zeros((ZR, D), jnp.float32)
    zeros_deg = jnp.zeros((ZR, DEG_W), jnp.float32)
    ones_rows = jnp.zeros((C, DEG_W), jnp.float32).at[:, 0].set(1.0)
    agg, deg = _sc_segment_sum(src, dst, feat, zeros_agg, zeros_deg,
                               ones_rows)
    return _tc_finish(agg, deg, h_e, W)


# speedup vs baseline: 8.6745x; 1.7324x over previous
import functools

import jax
import jax.numpy as jnp
from jax import lax
from jax.experimental import pallas as pl
from jax.experimental.pallas import tpu as pltpu
from jax.experimental.pallas import tpu_sc as plsc

N = 10000
NP = 10112
E = 320000
D = 128
DEG_W = 16

NC = 2
NS = 16
NW = NC * NS
EPW = E // NW
C = 80
CHUNKS = EPW // C
RPT = NP // NS
ZR = 79
ZCH = RPT // ZR


def _sc_segment_sum(idx_packed, feat, zeros_agg, zeros_deg, ones_rows):
    mesh = plsc.VectorSubcoreMesh(core_axis_name="c", subcore_axis_name="s")

    @functools.partial(
        pl.kernel,
        out_type=(
            jax.ShapeDtypeStruct((NC, NP, D), jnp.float32),
            jax.ShapeDtypeStruct((NC, NP, DEG_W), jnp.float32),
        ),
        mesh=mesh,
        compiler_params=pltpu.CompilerParams(use_tc_tiling_on_sc=False),
        scratch_types=[
            pltpu.VMEM_SHARED((NP, D), jnp.float32),
            pltpu.VMEM_SHARED((NP, DEG_W), jnp.float32),
            pltpu.VMEM((2, C), jnp.int32),
            pltpu.VMEM((2, C), jnp.int32),
            pltpu.VMEM((C, D), jnp.float32),
            pltpu.VMEM((C, D), jnp.float32),
            pltpu.VMEM((C, DEG_W), jnp.float32),
            pltpu.SemaphoreType.DMA,
            pltpu.SemaphoreType.DMA,
            pltpu.SemaphoreType.DMA,
            pltpu.SemaphoreType.DMA,
            pltpu.SemaphoreType.DMA,
            pltpu.SemaphoreType.DMA,
            pltpu.SemaphoreType.DMA,
            pltpu.SemaphoreType.DMA,
        ],
    )
    def k(idx_hbm, feat_hbm, zagg_hbm, zdeg_hbm, ones_hbm,
          agg_out, deg_out, agg_sp, deg_sp,
          idx0, idx1, rows0, rows1, ones_v,
          gsem0, gsem1, ssem0, ssem1, dsem0, dsem1, osem0, osem1):
        cid = lax.axis_index("c")
        sid = lax.axis_index("s")
        wid = sid * NC + cid
        r0 = sid * RPT
        cbase = wid * CHUNKS
        idx = (idx0, idx1)
        rows = (rows0, rows1)
        gsem = (gsem0, gsem1)
        ssem = (ssem0, ssem1)
        dsem = (dsem0, dsem1)
        osem = (osem0, osem1)

        pltpu.sync_copy(zagg_hbm, rows0.at[pl.ds(0, ZR)])
        pltpu.sync_copy(zdeg_hbm, ones_v.at[pl.ds(0, ZR)])
        for m in range(ZCH):
            pltpu.async_copy(rows0.at[pl.ds(0, ZR)],
                             agg_sp.at[pl.ds(r0 + m * ZR, ZR)], osem0)
            pltpu.async_copy(ones_v.at[pl.ds(0, ZR)],
                             deg_sp.at[pl.ds(r0 + m * ZR, ZR)], osem1)
        for m in range(ZCH):
            pltpu.make_async_copy(rows0.at[pl.ds(0, ZR)],
                                  agg_sp.at[pl.ds(r0 + m * ZR, ZR)],
                                  osem0).wait()
            pltpu.make_async_copy(ones_v.at[pl.ds(0, ZR)],
                                  deg_sp.at[pl.ds(r0 + m * ZR, ZR)],
                                  osem1).wait()
        pltpu.sync_copy(ones_hbm, ones_v)
        plsc.subcore_barrier()

        def load_idx(j, b):
            pltpu.sync_copy(idx_hbm.at[cbase + j], idx[b])

        def start_gather(b):
            pltpu.async_copy(feat_hbm.at[idx[b].at[0]], rows[b], gsem[b])

        def wait_gather(b):
            pltpu.make_async_copy(feat_hbm.at[idx[b].at[0]], rows[b],
                                  gsem[b]).wait()

        def start_scatter(b):
            pltpu.async_copy(rows[b], agg_sp.at[idx[b].at[1]], ssem[b],
                             add=True)
            pltpu.async_copy(ones_v, deg_sp.at[idx[b].at[1]], dsem[b],
                             add=True)

        def wait_scatter(b):
            pltpu.make_async_copy(rows[b], agg_sp.at[idx[b].at[1]],
                                  ssem[b]).wait()
            pltpu.make_async_copy(ones_v, deg_sp.at[idx[b].at[1]],
                                  dsem[b]).wait()

        load_idx(0, 0)
        start_gather(0)
        load_idx(1, 1)
        start_gather(1)
        wait_gather(0)
        start_scatter(0)

        def body(t, _):
            wait_scatter(0)
            load_idx(2 * t + 2, 0)
            start_gather(0)
            wait_gather(1)
            start_scatter(1)
            wait_scatter(1)

            @pl.when(2 * t + 3 < CHUNKS)
            def _():
                load_idx(2 * t + 3, 1)
                start_gather(1)

            wait_gather(0)
            start_scatter(0)
            return 0

        lax.fori_loop(0, (CHUNKS - 1) // 2, body, 0)
        wait_scatter(0)
        plsc.subcore_barrier()

        for m in range(ZCH):
            p = m % 2
            rr = r0 + m * ZR
            if m >= 2:
                rp = r0 + (m - 2) * ZR
                pltpu.make_async_copy(rows[p].at[pl.ds(0, ZR)],
                                      agg_out.at[cid, pl.ds(rp, ZR)],
                                      osem[p]).wait()
            pltpu.sync_copy(agg_sp.at[pl.ds(rr, ZR)],
                            rows[p].at[pl.ds(0, ZR)])
            pltpu.async_copy(rows[p].at[pl.ds(0, ZR)],
                             agg_out.at[cid, pl.ds(rr, ZR)], osem[p])
            pltpu.sync_copy(deg_sp.at[pl.ds(rr, ZR)],
                            ones_v.at[pl.ds(0, ZR)])
            pltpu.sync_copy(ones_v.at[pl.ds(0, ZR)],
                            deg_out.at[cid, pl.ds(rr, ZR)])
        for m in (ZCH - 2, ZCH - 1):
            p = m % 2
            rr = r0 + m * ZR
            pltpu.make_async_copy(rows[p].at[pl.ds(0, ZR)],
                                  agg_out.at[cid, pl.ds(rr, ZR)],
                                  osem[p]).wait()

    return k(idx_packed, feat, zeros_agg, zeros_deg, ones_rows)


def _tc_finish_body(agg_ref, deg_ref, he_ref, w_ref, out_ref):
    x = agg_ref[0] + agg_ref[1]
    x = x * he_ref[...]
    y = jnp.dot(x, w_ref[...], preferred_element_type=jnp.float32)
    deg = deg_ref[0, :, 0:1] + deg_ref[1, :, 0:1]
    out_ref[...] = y * (1.0 / jnp.maximum(deg, 1.0))


def _tc_finish(agg, deg, h_e, W):
    bn = 2000
    grid = (N // bn,)
    return pl.pallas_call(
        _tc_finish_body,
        grid=grid,
        in_specs=[
            pl.BlockSpec((NC, bn, D), lambda i: (0, i, 0)),
            pl.BlockSpec((NC, bn, DEG_W), lambda i: (0, i, 0)),
            pl.BlockSpec((1, D), lambda i: (0, 0)),
            pl.BlockSpec((D, D), lambda i: (0, 0)),
        ],
        out_specs=pl.BlockSpec((bn, D), lambda i: (i, 0)),
        out_shape=jax.ShapeDtypeStruct((N, D), jnp.float32),
    )(agg, deg, h_e, W)


def kernel(feat, edge_index, h_e, W):
    src = edge_index[0].reshape(NW, CHUNKS, C)
    dst = edge_index[1].reshape(NW, CHUNKS, C)
    idx_packed = jnp.stack([src, dst], axis=2).reshape(NW * CHUNKS, 2, C)
    zeros_agg = jnp.zeros((ZR, D), jnp.float32)
    zeros_deg = jnp.zeros((ZR, DEG_W), jnp.float32)
    ones_rows = jnp.zeros((C, DEG_W), jnp.float32).at[:, 0].set(1.0)
    agg, deg = _sc_segment_sum(idx_packed, feat, zeros_agg, zeros_deg,
                               ones_rows)
    return _tc_finish(agg, deg, h_e, W)


# speedup vs baseline: 9.5369x; 1.0994x over previous
import functools

import jax
import jax.numpy as jnp
from jax import lax
from jax.experimental import pallas as pl
from jax.experimental.pallas import tpu as pltpu
from jax.experimental.pallas import tpu_sc as plsc

N = 10000
NP = 10112
E = 320000
D = 128
DEG_W = 16

NC = 2
NS = 16
NW = NC * NS
EPW = E // NW
C = 80
CHUNKS = EPW // C
BI = 5
NB = CHUNKS // BI
RPT = NP // NS
ZR = 79
ZCH = RPT // ZR


def _sc_segment_sum(idx_packed, feat, zeros_agg, zeros_deg, ones_rows):
    mesh = plsc.VectorSubcoreMesh(core_axis_name="c", subcore_axis_name="s")

    @functools.partial(
        pl.kernel,
        out_type=(
            jax.ShapeDtypeStruct((NC, NP, D), jnp.float32),
            jax.ShapeDtypeStruct((NC, NP, DEG_W), jnp.float32),
        ),
        mesh=mesh,
        compiler_params=pltpu.CompilerParams(use_tc_tiling_on_sc=False),
        scratch_types=[
            pltpu.VMEM_SHARED((NP, D), jnp.float32),
            pltpu.VMEM_SHARED((NP, DEG_W), jnp.float32),
            pltpu.VMEM((BI, 2, C), jnp.int32),
            pltpu.VMEM((BI, 2, C), jnp.int32),
            pltpu.VMEM((C, D), jnp.float32),
            pltpu.VMEM((C, D), jnp.float32),
            pltpu.VMEM((C, DEG_W), jnp.float32),
            pltpu.SemaphoreType.DMA,
            pltpu.SemaphoreType.DMA,
            pltpu.SemaphoreType.DMA,
            pltpu.SemaphoreType.DMA,
            pltpu.SemaphoreType.DMA,
            pltpu.SemaphoreType.DMA,
            pltpu.SemaphoreType.DMA,
            pltpu.SemaphoreType.DMA,
            pltpu.SemaphoreType.DMA,
            pltpu.SemaphoreType.DMA,
        ],
    )
    def k(idx_hbm, feat_hbm, zagg_hbm, zdeg_hbm, ones_hbm,
          agg_out, deg_out, agg_sp, deg_sp,
          bidx0, bidx1, rows0, rows1, ones_v,
          gsem0, gsem1, ssem0, ssem1, dsem0, dsem1, osem0, osem1,
          bsem0, bsem1):
        cid = lax.axis_index("c")
        sid = lax.axis_index("s")
        wid = sid * NC + cid
        r0 = sid * RPT
        bbase = wid * NB
        bidx = (bidx0, bidx1)
        rows = (rows0, rows1)
        gsem = (gsem0, gsem1)
        ssem = (ssem0, ssem1)
        dsem = (dsem0, dsem1)
        osem = (osem0, osem1)
        bsem = (bsem0, bsem1)

        pltpu.sync_copy(zagg_hbm, rows0.at[pl.ds(0, ZR)])
        pltpu.sync_copy(zdeg_hbm, ones_v.at[pl.ds(0, ZR)])
        for m in range(ZCH):
            pltpu.async_copy(rows0.at[pl.ds(0, ZR)],
                             agg_sp.at[pl.ds(r0 + m * ZR, ZR)], osem0)
            pltpu.async_copy(ones_v.at[pl.ds(0, ZR)],
                             deg_sp.at[pl.ds(r0 + m * ZR, ZR)], osem1)
        for m in range(ZCH):
            pltpu.make_async_copy(rows0.at[pl.ds(0, ZR)],
                                  agg_sp.at[pl.ds(r0 + m * ZR, ZR)],
                                  osem0).wait()
            pltpu.make_async_copy(ones_v.at[pl.ds(0, ZR)],
                                  deg_sp.at[pl.ds(r0 + m * ZR, ZR)],
                                  osem1).wait()
        pltpu.sync_copy(ones_hbm, ones_v)
        plsc.subcore_barrier()

        def load_block(blk, P):
            pltpu.async_copy(idx_hbm.at[bbase + blk], bidx[P], bsem[P])

        def wait_block(P):
            pltpu.make_async_copy(idx_hbm.at[bbase], bidx[P],
                                  bsem[P]).wait()

        def start_gather(P, k, b):
            pltpu.async_copy(feat_hbm.at[bidx[P].at[k, 0]], rows[b],
                             gsem[b])

        def wait_gather(P, k, b):
            pltpu.make_async_copy(feat_hbm.at[bidx[P].at[k, 0]], rows[b],
                                  gsem[b]).wait()

        def start_scatter(P, k, b):
            pltpu.async_copy(rows[b], agg_sp.at[bidx[P].at[k, 1]],
                             ssem[b], add=True)
            pltpu.async_copy(ones_v, deg_sp.at[bidx[P].at[k, 1]],
                             dsem[b], add=True)

        def wait_scatter(P, k, b):
            pltpu.make_async_copy(rows[b], agg_sp.at[bidx[P].at[k, 1]],
                                  ssem[b]).wait()
            pltpu.make_async_copy(ones_v, deg_sp.at[bidx[P].at[k, 1]],
                                  dsem[b]).wait()

        def pkb(x):
            return ((x // BI) % 2, x % BI, x % 2)

        pltpu.sync_copy(idx_hbm.at[bbase], bidx[0])
        load_block(1, 1)
        start_gather(*pkb(0))
        start_gather(*pkb(1))
        wait_gather(*pkb(0))
        start_scatter(*pkb(0))
        wait_scatter(*pkb(0))
        start_gather(*pkb(2))
        wait_gather(*pkb(1))
        start_scatter(*pkb(1))

        def pkb_static(x):
            return pkb(x % 10)

        def pair_body(t, _):
            for m in range(10):
                wait_scatter(*pkb_static(1 + m))
                if m == 3:
                    load_block(2 * t + 2, 0)
                if m == 8:

                    @pl.when(t < (CHUNKS - 1) // 10 - 1)
                    def _():
                        load_block(2 * t + 3, 1)

                if m == 7:
                    wait_block(0)
                if m == 2:
                    wait_block(1)
                start_gather(*pkb_static(3 + m))
                wait_gather(*pkb_static(2 + m))
                start_scatter(*pkb_static(2 + m))
            return 0

        lax.fori_loop(0, (CHUNKS - 5) // 10, pair_body, 0)

        for j in (CHUNKS - 3, CHUNKS - 2):
            wait_scatter(*pkb_static(j - 1))
            start_gather(*pkb_static(j + 1))
            wait_gather(*pkb_static(j))
            start_scatter(*pkb_static(j))
        wait_scatter(*pkb_static(CHUNKS - 2))
        wait_gather(*pkb_static(CHUNKS - 1))
        start_scatter(*pkb_static(CHUNKS - 1))
        wait_scatter(*pkb_static(CHUNKS - 1))
        plsc.subcore_barrier()

        for m in range(ZCH):
            p = m % 2
            rr = r0 + m * ZR
            if m >= 2:
                rp = r0 + (m - 2) * ZR
                pltpu.make_async_copy(rows[p].at[pl.ds(0, ZR)],
                                      agg_out.at[cid, pl.ds(rp, ZR)],
                                      osem[p]).wait()
            pltpu.sync_copy(agg_sp.at[pl.ds(rr, ZR)],
                            rows[p].at[pl.ds(0, ZR)])
            pltpu.async_copy(rows[p].at[pl.ds(0, ZR)],
                             agg_out.at[cid, pl.ds(rr, ZR)], osem[p])
            pltpu.sync_copy(deg_sp.at[pl.ds(rr, ZR)],
                            ones_v.at[pl.ds(0, ZR)])
            pltpu.sync_copy(ones_v.at[pl.ds(0, ZR)],
                            deg_out.at[cid, pl.ds(rr, ZR)])
        for m in (ZCH - 2, ZCH - 1):
            p = m % 2
            rr = r0 + m * ZR
            pltpu.make_async_copy(rows[p].at[pl.ds(0, ZR)],
                                  agg_out.at[cid, pl.ds(rr, ZR)],
                                  osem[p]).wait()

    return k(idx_packed, feat, zeros_agg, zeros_deg, ones_rows)


def _tc_finish_body(agg_ref, deg_ref, he_ref, w_ref, out_ref):
    x = agg_ref[0] + agg_ref[1]
    x = x * he_ref[...]
    y = jnp.dot(x, w_ref[...], preferred_element_type=jnp.float32)
    deg = deg_ref[0, :, 0:1] + deg_ref[1, :, 0:1]
    out_ref[...] = y * (1.0 / jnp.maximum(deg, 1.0))


def _tc_finish(agg, deg, h_e, W):
    bn = 2000
    grid = (N // bn,)
    return pl.pallas_call(
        _tc_finish_body,
        grid=grid,
        in_specs=[
            pl.BlockSpec((NC, bn, D), lambda i: (0, i, 0)),
            pl.BlockSpec((NC, bn, DEG_W), lambda i: (0, i, 0)),
            pl.BlockSpec((1, D), lambda i: (0, 0)),
            pl.BlockSpec((D, D), lambda i: (0, 0)),
        ],
        out_specs=pl.BlockSpec((bn, D), lambda i: (i, 0)),
        out_shape=jax.ShapeDtypeStruct((N, D), jnp.float32),
    )(agg, deg, h_e, W)


def kernel(feat, edge_index, h_e, W):
    src = edge_index[0].reshape(NW, CHUNKS, C)
    dst = edge_index[1].reshape(NW, CHUNKS, C)
    idx_packed = jnp.stack([src, dst], axis=2).reshape(NW * NB, BI, 2, C)
    zeros_agg = jnp.zeros((ZR, D), jnp.float32)
    zeros_deg = jnp.zeros((ZR, DEG_W), jnp.float32)
    ones_rows = jnp.zeros((C, DEG_W), jnp.float32).at[:, 0].set(1.0)
    agg, deg = _sc_segment_sum(idx_packed, feat, zeros_agg, zeros_deg,
                               ones_rows)
    return _tc_finish(agg, deg, h_e, W)


# speedup vs baseline: 11.6535x; 1.2219x over previous
import functools

import jax
import jax.numpy as jnp
import numpy as np
from jax import lax
from jax.experimental import pallas as pl
from jax.experimental.pallas import tpu as pltpu
from jax.experimental.pallas import tpu_sc as plsc

N = 10000
NP = 10112
E = 320000
D = 128
DEG_W = 16

NC = 2
NS = 16
NW = NC * NS
EPW = E // NW
C = 80
CHUNKS = EPW // C
BI = 5
NB = CHUNKS // BI
RPT = NP // NS
ZR = 79
ZCH = RPT // ZR


def _sc_segment_sum(ei, feat, zeros_agg, zeros_deg, ones_rows):
    mesh = plsc.VectorSubcoreMesh(core_axis_name="c", subcore_axis_name="s")

    @functools.partial(
        pl.kernel,
        out_type=(
            jax.ShapeDtypeStruct((NC, NP, D), jnp.float32),
            jax.ShapeDtypeStruct((NC, NP, DEG_W), jnp.float32),
        ),
        mesh=mesh,
        compiler_params=pltpu.CompilerParams(use_tc_tiling_on_sc=False),
        scratch_types=[
            pltpu.VMEM_SHARED((NP, D), jnp.float32),
            pltpu.VMEM_SHARED((NP, DEG_W), jnp.float32),
            pltpu.VMEM((BI, C), jnp.int32),
            pltpu.VMEM((BI, C), jnp.int32),
            pltpu.VMEM((BI, C), jnp.int32),
            pltpu.VMEM((BI, C), jnp.int32),
            pltpu.VMEM((C, D), jnp.float32),
            pltpu.VMEM((C, D), jnp.float32),
            pltpu.VMEM((C, DEG_W), jnp.float32),
            pltpu.SemaphoreType.DMA,
            pltpu.SemaphoreType.DMA,
            pltpu.SemaphoreType.DMA,
            pltpu.SemaphoreType.DMA,
            pltpu.SemaphoreType.DMA,
            pltpu.SemaphoreType.DMA,
            pltpu.SemaphoreType.DMA,
            pltpu.SemaphoreType.DMA,
            pltpu.SemaphoreType.DMA,
            pltpu.SemaphoreType.DMA,
        ],
    )
    def k(ei_hbm, feat_hbm, zagg_hbm, zdeg_hbm, ones_hbm,
          agg_out, deg_out, agg_sp, deg_sp,
          bsrc0, bsrc1, bdst0, bdst1, rows0, rows1, ones_v,
          gsem0, gsem1, ssem0, ssem1, dsem0, dsem1, osem0, osem1,
          bsem0, bsem1):
        cid = lax.axis_index("c")
        sid = lax.axis_index("s")
        wid = sid * NC + cid
        r0 = sid * RPT
        bsrc = (bsrc0, bsrc1)
        bdst = (bdst0, bdst1)
        rows = (rows0, rows1)
        gsem = (gsem0, gsem1)
        ssem = (ssem0, ssem1)
        dsem = (dsem0, dsem1)
        osem = (osem0, osem1)
        bsem = (bsem0, bsem1)

        pltpu.sync_copy(zagg_hbm, rows0.at[pl.ds(0, ZR)])
        pltpu.sync_copy(zdeg_hbm, ones_v.at[pl.ds(0, ZR)])
        for m in range(ZCH):
            pltpu.async_copy(rows0.at[pl.ds(0, ZR)],
                             agg_sp.at[pl.ds(r0 + m * ZR, ZR)], osem0)
            pltpu.async_copy(ones_v.at[pl.ds(0, ZR)],
                             deg_sp.at[pl.ds(r0 + m * ZR, ZR)], osem1)
        for m in range(ZCH):
            pltpu.make_async_copy(rows0.at[pl.ds(0, ZR)],
                                  agg_sp.at[pl.ds(r0 + m * ZR, ZR)],
                                  osem0).wait()
            pltpu.make_async_copy(ones_v.at[pl.ds(0, ZR)],
                                  deg_sp.at[pl.ds(r0 + m * ZR, ZR)],
                                  osem1).wait()
        pltpu.sync_copy(ones_hbm, ones_v)
        plsc.subcore_barrier()

        def load_block(blk, P):
            pltpu.async_copy(ei_hbm.at[0, wid, blk], bsrc[P], bsem[P])
            pltpu.async_copy(ei_hbm.at[1, wid, blk], bdst[P], bsem[P])

        def wait_block(P):
            pltpu.make_async_copy(ei_hbm.at[0, wid, 0], bsrc[P],
                                  bsem[P]).wait()
            pltpu.make_async_copy(ei_hbm.at[1, wid, 0], bdst[P],
                                  bsem[P]).wait()

        def start_gather(P, k, b):
            pltpu.async_copy(feat_hbm.at[bsrc[P].at[k]], rows[b],
                             gsem[b])

        def wait_gather(P, k, b):
            pltpu.make_async_copy(feat_hbm.at[bsrc[P].at[k]], rows[b],
                                  gsem[b]).wait()

        def start_scatter(P, k, b):
            pltpu.async_copy(rows[b], agg_sp.at[bdst[P].at[k]],
                             ssem[b], add=True)
            pltpu.async_copy(ones_v, deg_sp.at[bdst[P].at[k]],
                             dsem[b], add=True)

        def wait_scatter(P, k, b):
            pltpu.make_async_copy(rows[b], agg_sp.at[bdst[P].at[k]],
                                  ssem[b]).wait()
            pltpu.make_async_copy(ones_v, deg_sp.at[bdst[P].at[k]],
                                  dsem[b]).wait()

        def pkb(x):
            return ((x // BI) % 2, x % BI, x % 2)

        pltpu.sync_copy(ei_hbm.at[0, wid, 0], bsrc[0])
        pltpu.sync_copy(ei_hbm.at[1, wid, 0], bdst[0])
        load_block(1, 1)
        start_gather(*pkb(0))
        start_gather(*pkb(1))
        wait_gather(*pkb(0))
        start_scatter(*pkb(0))
        wait_scatter(*pkb(0))
        start_gather(*pkb(2))
        wait_gather(*pkb(1))
        start_scatter(*pkb(1))

        def pkb_static(x):
            return pkb(x % 10)

        def pair_body(t, _):
            for m in range(10):
                wait_scatter(*pkb_static(1 + m))
                if m == 3:
                    load_block(2 * t + 2, 0)
                if m == 8:

                    @pl.when(t < (CHUNKS - 1) // 10 - 1)
                    def _():
                        load_block(2 * t + 3, 1)

                if m == 7:
                    wait_block(0)
                if m == 2:
                    wait_block(1)
                start_gather(*pkb_static(3 + m))
                wait_gather(*pkb_static(2 + m))
                start_scatter(*pkb_static(2 + m))
            return 0

        lax.fori_loop(0, (CHUNKS - 5) // 10, pair_body, 0)

        for j in (CHUNKS - 3, CHUNKS - 2):
            wait_scatter(*pkb_static(j - 1))
            start_gather(*pkb_static(j + 1))
            wait_gather(*pkb_static(j))
            start_scatter(*pkb_static(j))
        wait_scatter(*pkb_static(CHUNKS - 2))
        wait_gather(*pkb_static(CHUNKS - 1))
        start_scatter(*pkb_static(CHUNKS - 1))
        wait_scatter(*pkb_static(CHUNKS - 1))
        plsc.subcore_barrier()

        for m in range(ZCH):
            p = m % 2
            rr = r0 + m * ZR
            if m >= 2:
                rp = r0 + (m - 2) * ZR
                pltpu.make_async_copy(rows[p].at[pl.ds(0, ZR)],
                                      agg_out.at[cid, pl.ds(rp, ZR)],
                                      osem[p]).wait()
            pltpu.sync_copy(agg_sp.at[pl.ds(rr, ZR)],
                            rows[p].at[pl.ds(0, ZR)])
            pltpu.async_copy(rows[p].at[pl.ds(0, ZR)],
                             agg_out.at[cid, pl.ds(rr, ZR)], osem[p])
            pltpu.sync_copy(deg_sp.at[pl.ds(rr, ZR)],
                            ones_v.at[pl.ds(0, ZR)])
            pltpu.sync_copy(ones_v.at[pl.ds(0, ZR)],
                            deg_out.at[cid, pl.ds(rr, ZR)])
        for m in (ZCH - 2, ZCH - 1):
            p = m % 2
            rr = r0 + m * ZR
            pltpu.make_async_copy(rows[p].at[pl.ds(0, ZR)],
                                  agg_out.at[cid, pl.ds(rr, ZR)],
                                  osem[p]).wait()

    return k(ei, feat, zeros_agg, zeros_deg, ones_rows)


def _tc_finish_body(agg_ref, deg_ref, he_ref, w_ref, out_ref):
    x = agg_ref[0] + agg_ref[1]
    x = x * he_ref[...]
    y = jnp.dot(x, w_ref[...], preferred_element_type=jnp.float32)
    deg = deg_ref[0, :, 0:1] + deg_ref[1, :, 0:1]
    out_ref[...] = y * (1.0 / jnp.maximum(deg, 1.0))


def _tc_finish(agg, deg, h_e, W):
    bn = 2000
    grid = (N // bn,)
    return pl.pallas_call(
        _tc_finish_body,
        grid=grid,
        in_specs=[
            pl.BlockSpec((NC, bn, D), lambda i: (0, i, 0)),
            pl.BlockSpec((NC, bn, DEG_W), lambda i: (0, i, 0)),
            pl.BlockSpec((1, D), lambda i: (0, 0)),
            pl.BlockSpec((D, D), lambda i: (0, 0)),
        ],
        out_specs=pl.BlockSpec((bn, D), lambda i: (i, 0)),
        out_shape=jax.ShapeDtypeStruct((N, D), jnp.float32),
    )(agg, deg, h_e, W)


_ZAGG = np.zeros((ZR, D), np.float32)
_ZDEG = np.zeros((ZR, DEG_W), np.float32)
_ONES = np.zeros((C, DEG_W), np.float32)
_ONES[:, 0] = 1.0


def kernel(feat, edge_index, h_e, W):
    ei = edge_index.reshape(2, NW, NB, BI, C)
    agg, deg = _sc_segment_sum(ei, feat, jnp.asarray(_ZAGG),
                               jnp.asarray(_ZDEG), jnp.asarray(_ONES))
    return _tc_finish(agg, deg, h_e, W)
